# baseline (speedup 1.0000x reference)
import functools

import jax
import jax.numpy as jnp
from jax import lax
from jax.experimental import pallas as pl
from jax.experimental.pallas import tpu as pltpu
from jax.experimental.pallas import tpu_sc as plsc

N = 10000
NP = 10240
D = 128
E = 320000
NC = 2
NS = 16
NW = NC * NS
K = 128
CH = 80
EP = NW * CH * K
RPT = NP // NS
WS = 16
KS = 512
CHS = EP // NW // KS
HD = D // 2
KR = 128
CHR = EP // NS // KR
NB = 4


def _sc_mesh():
    return plsc.VectorSubcoreMesh(core_axis_name="c", subcore_axis_name="s",
                                  num_cores=NC, num_subcores=NS)


def _edge_pipeline(mk_src, acc, di2, bufs, sems, nchunks):
    nb = len(bufs)
    rounds = nchunks // nb

    def gather(c, buf, sem):
        pltpu.async_copy(mk_src(c), buf, sem)

    def gather_wait(c, buf, sem):
        pltpu.make_async_copy(mk_src(c), buf, sem).wait()

    for b in range(nb):
        gather(b, bufs[b], sems[b])

    def body(j, carry):
        base = nb * j
        scats = []
        for b in range(nb):
            gather_wait(base + b, bufs[b], sems[b])
            scats.append(pltpu.async_copy(
                bufs[b], acc.at[di2.at[base + b]], sems[b], add=True))
        for b in range(nb):
            scats[b].wait()

            @pl.when(j < rounds - 1)
            def _(b=b):
                gather(base + nb + b, bufs[b], sems[b])

        return carry

    lax.fori_loop(0, rounds, body, 0)


def _newton_rsqrt16(v):
    i = plsc.bitcast(v, jnp.int32)
    i = jnp.int32(0x5F3759DF) - lax.shift_right_logical(i, 1)
    y = plsc.bitcast(i, jnp.float32)
    for _ in range(3):
        y = y * (1.5 - 0.5 * v * y * y)
    return y


@functools.partial(
    pl.kernel,
    out_type=[
        jax.ShapeDtypeStruct((NC, NP, HD), jnp.float32),
        jax.ShapeDtypeStruct((NC, NP, HD), jnp.float32),
        jax.ShapeDtypeStruct((NC, NP, WS), jnp.float32),
    ],
    mesh=_sc_mesh(),
    scratch_types=[
        pltpu.VMEM((CHR // 2, KR), jnp.int32),
        pltpu.VMEM((CHR // 2, KR), jnp.int32),
        [pltpu.VMEM((KR, HD), jnp.float32)] * NB,
        pltpu.VMEM((KR, WS), jnp.float32),
        pltpu.VMEM((RPT, WS), jnp.float32),
        pltpu.VMEM_SHARED((NP, HD), jnp.float32),
        pltpu.VMEM_SHARED((NP, WS), jnp.float32),
        [pltpu.SemaphoreType.DMA] * NB,
        pltpu.SemaphoreType.DMA,
    ],
    compiler_params=pltpu.CompilerParams(use_tc_tiling_on_sc=False,
                                         needs_layout_passes=False),
)
def _mega(x2_hbm, src_hbm, dst_hbm, ones_hbm, zrow_hbm, zrow8_hbm,
          agg_hbm, z_hbm, dinv_hbm,
          si2, di2, bufs, ones, dv16, acc, acc8, sems, semd):
    cid = lax.axis_index("c")
    sid = lax.axis_index("s")
    hch = CHR // 2
    pltpu.sync_copy(ones_hbm, ones)
    pltpu.sync_copy(zrow_hbm, acc.at[pl.ds(sid * RPT, RPT)])
    pltpu.sync_copy(zrow8_hbm, acc8.at[pl.ds(sid * RPT, RPT)])
    plsc.subcore_barrier()

    def fire(c, carry):
        pltpu.async_copy(ones, acc8.at[di2.at[c]], semd, add=True)
        return carry

    def drain(c, carry):
        pltpu.make_async_copy(ones, acc8.at[di2.at[c]], semd).wait()
        return carry

    for p in range(2):
        pltpu.sync_copy(dst_hbm.at[sid, pl.ds(p * hch, hch)], di2)
        lax.fori_loop(0, hch, fire, 0)
        lax.fori_loop(0, hch, drain, 0)
    plsc.subcore_barrier()

    pltpu.sync_copy(acc8.at[pl.ds(sid * RPT, RPT)], dv16)

    def newton(r, carry):
        dv16[r] = _newton_rsqrt16(dv16[r] + 1.0)
        return carry

    lax.fori_loop(0, RPT, newton, 0)
    pltpu.sync_copy(dv16, dinv_hbm.at[cid, pl.ds(sid * RPT, RPT)])

    for k in range(RPT // KR):
        base = sid * RPT + k * KR
        pltpu.sync_copy(x2_hbm.at[cid, pl.ds(base, KR)], bufs[0])

        def scale(r, carry):
            s = dv16[k * KR + r][0]
            for j in range(HD // 16):
                bufs[0][r, pl.ds(j * 16, 16)] = \
                    bufs[0][r, pl.ds(j * 16, 16)] * s
            return carry

        lax.fori_loop(0, KR, scale, 0)
        pltpu.sync_copy(bufs[0], z_hbm.at[cid, pl.ds(base, KR)])
    plsc.subcore_barrier()

    def mk_src(c):
        return z_hbm.at[cid].at[si2.at[c]]

    for p in range(2):
        pltpu.sync_copy(src_hbm.at[sid, pl.ds(p * hch, hch)], si2)
        pltpu.sync_copy(dst_hbm.at[sid, pl.ds(p * hch, hch)], di2)
        _edge_pipeline(mk_src, acc, di2, bufs, sems, hch)
    plsc.subcore_barrier()
    pltpu.sync_copy(acc.at[pl.ds(sid * RPT, RPT)],
                    agg_hbm.at[cid, pl.ds(sid * RPT, RPT)])


@functools.partial(
    pl.kernel,
    out_type=jax.ShapeDtypeStruct((NC, NP, WS), jnp.float32),
    mesh=_sc_mesh(),
    scratch_types=[
        pltpu.VMEM((CHS, KS), jnp.int32),
        pltpu.VMEM((CHS, KS), jnp.int32),
        [pltpu.VMEM((KS, WS), jnp.float32)] * NB,
        pltpu.VMEM_SHARED((NP, WS), jnp.float32),
        [pltpu.SemaphoreType.DMA] * NB,
    ],
    compiler_params=pltpu.CompilerParams(use_tc_tiling_on_sc=False),
)
def _scalar_agg(tab_hbm, src_hbm, dst_hbm, zrow_hbm, out_hbm,
                si2, di2, bufs, acc, sems):
    cid = lax.axis_index("c")
    sid = lax.axis_index("s")
    wid = sid * NC + cid
    pltpu.sync_copy(src_hbm.at[wid], si2)
    pltpu.sync_copy(dst_hbm.at[wid], di2)
    pltpu.sync_copy(zrow_hbm, acc.at[pl.ds(sid * RPT, RPT)])
    plsc.subcore_barrier()

    def mk_src(c):
        return tab_hbm.at[si2.at[c]]

    _edge_pipeline(mk_src, acc, di2, bufs, sems, CHS)
    plsc.subcore_barrier()
    pltpu.sync_copy(acc.at[pl.ds(sid * RPT, RPT)],
                    out_hbm.at[cid, pl.ds(sid * RPT, RPT)])


def _tc_d_body(agg, z, dinv, w1, b1, w2, t_out):
    g = jnp.dot(agg[...] + z[...], w1[...], preferred_element_type=jnp.float32)
    h = jnp.maximum(g * dinv[...] + b1[...], 0.0)
    s = jnp.dot(h, w2[...], preferred_element_type=jnp.float32)
    t_out[...] = jnp.broadcast_to(s * dinv[...], (N, WS))


_tc_d = pl.pallas_call(
    _tc_d_body,
    out_shape=jax.ShapeDtypeStruct((N, WS), jnp.float32),
)


def _tc_f_body(acc0, acc1, t, dinv, b2, out):
    o = jax.nn.sigmoid((acc0[...] + acc1[...] + t[...]) * dinv[...] + b2[...])
    out[...] = (jnp.sum(o) / N).reshape(1, 1)


_tc_f = pl.pallas_call(
    _tc_f_body,
    out_shape=jax.ShapeDtypeStruct((1, 1), jnp.float32),
)


def kernel(x, edge_index, W1, b1, W2, b2):
    f32 = jnp.float32
    src = edge_index[0].astype(jnp.int32)
    dst = edge_index[1].astype(jnp.int32)

    pad = jnp.arange(EP - E, dtype=jnp.int32)
    src_p = jnp.concatenate([src, pad % N])
    dst_p = jnp.concatenate([dst, N + pad % (NP - N)])
    srcR = src_p.reshape(NS, CHR, KR)
    dstR = dst_p.reshape(NS, CHR, KR)
    srcS = src_p.reshape(NW, CHS, KS)
    dstS = dst_p.reshape(NW, CHS, KS)

    onesK = jnp.ones((KR, WS), f32)
    zrow_h = jnp.zeros((RPT, HD), f32)
    zrow_w = jnp.zeros((RPT, WS), f32)
    x2 = jnp.stack([x[:, :HD], x[:, HD:]])
    x2 = jnp.pad(x2, ((0, 0), (0, NP - N), (0, 0)))

    aggp, z2, dinv16 = _mega(x2, srcR, dstR, onesK, zrow_h, zrow_w)
    agg = jnp.concatenate([aggp[0, :N], aggp[1, :N]], axis=1)
    z = jnp.concatenate([z2[0, :N], z2[1, :N]], axis=1)
    dinv = dinv16[0, :N, 0:1]
    t16 = _tc_d(agg, z, dinv, W1, b1.reshape(1, D), W2)

    accp = _scalar_agg(t16, srcS, dstS, zrow_w)
    out = _tc_f(accp[0, :N, 0:1], accp[1, :N, 0:1],
                t16[:, 0:1], dinv, b2.reshape(1, 1))
    return out.reshape(1)


# speedup vs baseline: 1.1749x; 1.1749x over previous
import functools

import jax
import jax.numpy as jnp
from jax import lax
from jax.experimental import pallas as pl
from jax.experimental.pallas import tpu as pltpu
from jax.experimental.pallas import tpu_sc as plsc

N = 10000
NP = 10240
D = 128
E = 320000
NC = 2
NS = 16
NW = NC * NS
K = 128
CH = 80
EP = NW * CH * K
RPT = NP // NS
WS = 16
KS = 512
CHS = EP // NW // KS
HD = D // 2
KR = 128
CHR = EP // NS // KR
NB = 5


def _sc_mesh():
    return plsc.VectorSubcoreMesh(core_axis_name="c", subcore_axis_name="s",
                                  num_cores=NC, num_subcores=NS)


def _edge_pipeline(mk_src, acc, di2, bufs, sems, nchunks):
    nb = len(bufs)
    rounds = nchunks // nb

    def gather(c, buf, sem):
        pltpu.async_copy(mk_src(c), buf, sem)

    def gather_wait(c, buf, sem):
        pltpu.make_async_copy(mk_src(c), buf, sem).wait()

    for b in range(nb):
        gather(b, bufs[b], sems[b])

    def body(j, carry):
        base = nb * j
        scats = []
        for b in range(nb):
            gather_wait(base + b, bufs[b], sems[b])
            scats.append(pltpu.async_copy(
                bufs[b], acc.at[di2.at[base + b]], sems[b], add=True))
        for b in range(nb):
            scats[b].wait()

            @pl.when(j < rounds - 1)
            def _(b=b):
                gather(base + nb + b, bufs[b], sems[b])

        return carry

    lax.fori_loop(0, rounds, body, 0)


@functools.partial(
    pl.kernel,
    out_type=jax.ShapeDtypeStruct((NC, NP, HD), jnp.float32),
    mesh=_sc_mesh(),
    scratch_types=[
        pltpu.VMEM((CHR, KR), jnp.int32),
        pltpu.VMEM((CHR, KR), jnp.int32),
        [pltpu.VMEM((KR, HD), jnp.float32)] * NB,
        pltpu.VMEM_SHARED((NP, HD), jnp.float32),
        [pltpu.SemaphoreType.DMA] * NB,
    ],
    compiler_params=pltpu.CompilerParams(use_tc_tiling_on_sc=False),
)
def _row_agg(y_hbm, src_hbm, dst_hbm, zrow_hbm, out_hbm,
             si2, di2, bufs, acc, sems):
    cid = lax.axis_index("c")
    sid = lax.axis_index("s")
    pltpu.sync_copy(src_hbm.at[sid], si2)
    pltpu.sync_copy(dst_hbm.at[sid], di2)
    pltpu.sync_copy(zrow_hbm, acc.at[pl.ds(sid * RPT, RPT)])
    plsc.subcore_barrier()

    def mk_src(c):
        return y_hbm.at[cid].at[si2.at[c]]

    _edge_pipeline(mk_src, acc, di2, bufs, sems, CHR)
    plsc.subcore_barrier()
    pltpu.sync_copy(acc.at[pl.ds(sid * RPT, RPT)],
                    out_hbm.at[cid, pl.ds(sid * RPT, RPT)])


@functools.partial(
    pl.kernel,
    out_type=jax.ShapeDtypeStruct((NC, NP, WS), jnp.float32),
    mesh=_sc_mesh(),
    scratch_types=[
        pltpu.VMEM((CHS, KS), jnp.int32),
        pltpu.VMEM((CHS, KS), jnp.int32),
        [pltpu.VMEM((KS, WS), jnp.float32)] * NB,
        pltpu.VMEM_SHARED((NP, WS), jnp.float32),
        [pltpu.SemaphoreType.DMA] * NB,
    ],
    compiler_params=pltpu.CompilerParams(use_tc_tiling_on_sc=False),
)
def _scalar_agg(tab_hbm, src_hbm, dst_hbm, zrow_hbm, out_hbm,
                si2, di2, bufs, acc, sems):
    cid = lax.axis_index("c")
    sid = lax.axis_index("s")
    wid = sid * NC + cid
    pltpu.sync_copy(src_hbm.at[wid], si2)
    pltpu.sync_copy(dst_hbm.at[wid], di2)
    pltpu.sync_copy(zrow_hbm, acc.at[pl.ds(sid * RPT, RPT)])
    plsc.subcore_barrier()

    def mk_src(c):
        return tab_hbm.at[si2.at[c]]

    _edge_pipeline(mk_src, acc, di2, bufs, sems, CHS)
    plsc.subcore_barrier()
    pltpu.sync_copy(acc.at[pl.ds(sid * RPT, RPT)],
                    out_hbm.at[cid, pl.ds(sid * RPT, RPT)])


@functools.partial(
    pl.kernel,
    out_type=jax.ShapeDtypeStruct((NC, NP, WS), jnp.float32),
    mesh=_sc_mesh(),
    scratch_types=[
        pltpu.VMEM((CHS, KS), jnp.int32),
        pltpu.VMEM((KS, WS), jnp.float32),
        pltpu.VMEM_SHARED((NP, WS), jnp.float32),
        pltpu.SemaphoreType.DMA,
    ],
    compiler_params=pltpu.CompilerParams(use_tc_tiling_on_sc=False),
)
def _deg_agg(ones_hbm, dst_hbm, zrow_hbm, out_hbm, di2, vals, acc, sem):
    cid = lax.axis_index("c")
    sid = lax.axis_index("s")
    wid = sid * NC + cid
    pltpu.sync_copy(ones_hbm, vals)
    pltpu.sync_copy(dst_hbm.at[wid], di2)
    pltpu.sync_copy(zrow_hbm, acc.at[pl.ds(sid * RPT, RPT)])
    plsc.subcore_barrier()

    def fire(c, carry):
        pltpu.async_copy(vals, acc.at[di2.at[c]], sem, add=True)
        return carry

    lax.fori_loop(0, CHS, fire, 0)

    def drain(c, carry):
        pltpu.make_async_copy(vals, acc.at[di2.at[c]], sem).wait()
        return carry

    lax.fori_loop(0, CHS, drain, 0)
    plsc.subcore_barrier()
    pltpu.sync_copy(acc.at[pl.ds(sid * RPT, RPT)],
                    out_hbm.at[cid, pl.ds(sid * RPT, RPT)])


def _tc_b_body(deg0, deg1, x, w1, y_out, y2_out, dinv_out):
    dinv = lax.rsqrt(deg0[...] + deg1[...] + 1.0)
    xw = jnp.dot(x[...], w1[...], preferred_element_type=jnp.float32)
    y = xw * dinv
    y_out[...] = y
    y2_out[...] = jnp.stack([y[:, :HD], y[:, HD:]])
    dinv_out[...] = dinv


_tc_b = pl.pallas_call(
    _tc_b_body,
    out_shape=[
        jax.ShapeDtypeStruct((N, D), jnp.float32),
        jax.ShapeDtypeStruct((NC, N, HD), jnp.float32),
        jax.ShapeDtypeStruct((N, 1), jnp.float32),
    ],
)


def _tc_d_body(agg, y, dinv, b1, w2, t_out):
    h = jnp.maximum((agg[...] + y[...]) * dinv[...] + b1[...], 0.0)
    s = jnp.dot(h, w2[...], preferred_element_type=jnp.float32)
    t_out[...] = jnp.broadcast_to(s * dinv[...], (N, WS))


_tc_d = pl.pallas_call(
    _tc_d_body,
    out_shape=jax.ShapeDtypeStruct((N, WS), jnp.float32),
)


def _tc_f_body(acc0, acc1, t, dinv, b2, out):
    o = jax.nn.sigmoid((acc0[...] + acc1[...] + t[...]) * dinv[...] + b2[...])
    out[...] = (jnp.sum(o) / N).reshape(1, 1)


_tc_f = pl.pallas_call(
    _tc_f_body,
    out_shape=jax.ShapeDtypeStruct((1, 1), jnp.float32),
)


def kernel(x, edge_index, W1, b1, W2, b2):
    f32 = jnp.float32
    src = edge_index[0].astype(jnp.int32)
    dst = edge_index[1].astype(jnp.int32)

    pad = jnp.arange(EP - E, dtype=jnp.int32)
    src_p = jnp.concatenate([src, pad % N])
    dst_p = jnp.concatenate([dst, N + pad % (NP - N)])
    srcR = src_p.reshape(NS, CHR, KR)
    dstR = dst_p.reshape(NS, CHR, KR)
    srcS = src_p.reshape(NW, CHS, KS)
    dstS = dst_p.reshape(NW, CHS, KS)

    onesK = jnp.ones((KS, WS), f32)
    zrow_h = jnp.zeros((RPT, HD), f32)
    zrow_w = jnp.zeros((RPT, WS), f32)

    degp = _deg_agg(onesK, dstS, zrow_w)
    y, y2, dinv = _tc_b(degp[0, :N, 0:1], degp[1, :N, 0:1], x, W1)

    aggp = _row_agg(y2, srcR, dstR, zrow_h)
    agg = jnp.concatenate([aggp[0, :N], aggp[1, :N]], axis=1)
    t16 = _tc_d(agg, y, dinv, b1.reshape(1, D), W2)

    accp = _scalar_agg(t16, srcS, dstS, zrow_w)
    out = _tc_f(accp[0, :N, 0:1], accp[1, :N, 0:1],
                t16[:, 0:1], dinv, b2.reshape(1, 1))
    return out.reshape(1)
